# Initial kernel scaffold; baseline (speedup 1.0000x reference)
import jax
import jax.numpy as jnp
from jax.experimental import pallas as pl


def kernel(x, edge_index, batch, W0, b0, W_rest, b_rest, lin_W, lin_b):
    raise NotImplementedError("write your pallas kernel here")


# speedup vs baseline: 1.0420x; 1.0420x over previous
import functools

import jax
import jax.numpy as jnp
from jax.experimental import pallas as pl


def _layer0_body(p_ref, x_ref, sw_ref, w_ref, b_ref, o_ref):
    s = p_ref[...] + x_ref[...] * sw_ref[...]
    v = s * w_ref[...] + b_ref[...]
    o_ref[...] = jnp.where(v > 0, v, jnp.exp(v) - 1.0)


def _layer_body(p_ref, h_ref, sw_ref, w_ref, b_ref, o_ref):
    s = p_ref[...] + h_ref[...] * sw_ref[...]
    v = jnp.dot(s, w_ref[...], preferred_element_type=jnp.float32) + b_ref[...]
    o_ref[...] = jnp.where(v > 0, v, jnp.exp(v) - 1.0)


def _pool_body(h_ref, seg_ref, sums_ref, cnt_ref, *, num_graphs):
    @pl.when(pl.program_id(0) == 0)
    def _init():
        sums_ref[...] = jnp.zeros_like(sums_ref)
        cnt_ref[...] = jnp.zeros_like(cnt_ref)

    seg = seg_ref[...][:, 0]
    iota = jax.lax.broadcasted_iota(jnp.int32, (num_graphs, seg.shape[0]), 0)
    onehot = (seg[None, :] == iota).astype(jnp.float32)
    sums_ref[...] += jnp.dot(onehot, h_ref[...],
                             preferred_element_type=jnp.float32)
    cnt_ref[...] += jnp.sum(onehot, axis=1, keepdims=True)


def _head_body(sums_ref, cnt_ref, w_ref, b_ref, o_ref):
    pooled = sums_ref[...] / jnp.maximum(cnt_ref[...], 1.0)
    o_ref[...] = (jnp.sum(pooled * w_ref[...][:, 0][None, :], axis=1,
                          keepdims=True) + b_ref[...])


def _pick_block(n):
    for bn in (2000, 2500, 1000, 500, 200, 100, 50, 20, 10, 5, 4, 2):
        if n % bn == 0:
            return bn
    return 1


def kernel(x, edge_index, batch, W0, b0, W_rest, b_rest, lin_W, lin_b):
    n = x.shape[0]
    hidden = W0.shape[1]
    num_graphs = 64
    f32 = jnp.float32

    src = edge_index[0].astype(jnp.int32)
    dst = edge_index[1].astype(jnp.int32)
    seg = batch.astype(jnp.int32)[:, None]

    deg = jnp.ones((n,), f32).at[dst].add(1.0)
    dinv = jax.lax.rsqrt(deg)
    norm_e = dinv[src] * dinv[dst]
    selfw = (dinv * dinv)[:, None]

    bn = _pick_block(n)
    grid = n // bn

    layer0 = pl.pallas_call(
        _layer0_body,
        grid=(grid,),
        in_specs=[
            pl.BlockSpec((bn, 1), lambda i: (i, 0)),
            pl.BlockSpec((bn, 1), lambda i: (i, 0)),
            pl.BlockSpec((bn, 1), lambda i: (i, 0)),
            pl.BlockSpec((1, hidden), lambda i: (0, 0)),
            pl.BlockSpec((1, hidden), lambda i: (0, 0)),
        ],
        out_specs=pl.BlockSpec((bn, hidden), lambda i: (i, 0)),
        out_shape=jax.ShapeDtypeStruct((n, hidden), f32),
    )

    layer = pl.pallas_call(
        _layer_body,
        grid=(grid,),
        in_specs=[
            pl.BlockSpec((bn, hidden), lambda i: (i, 0)),
            pl.BlockSpec((bn, hidden), lambda i: (i, 0)),
            pl.BlockSpec((bn, 1), lambda i: (i, 0)),
            pl.BlockSpec((hidden, hidden), lambda i: (0, 0)),
            pl.BlockSpec((1, hidden), lambda i: (0, 0)),
        ],
        out_specs=pl.BlockSpec((bn, hidden), lambda i: (i, 0)),
        out_shape=jax.ShapeDtypeStruct((n, hidden), f32),
    )

    pool = pl.pallas_call(
        functools.partial(_pool_body, num_graphs=num_graphs),
        grid=(grid,),
        in_specs=[
            pl.BlockSpec((bn, hidden), lambda i: (i, 0)),
            pl.BlockSpec((bn, 1), lambda i: (i, 0)),
        ],
        out_specs=[
            pl.BlockSpec((num_graphs, hidden), lambda i: (0, 0)),
            pl.BlockSpec((num_graphs, 1), lambda i: (0, 0)),
        ],
        out_shape=[
            jax.ShapeDtypeStruct((num_graphs, hidden), f32),
            jax.ShapeDtypeStruct((num_graphs, 1), f32),
        ],
    )

    head = pl.pallas_call(
        _head_body,
        in_specs=[
            pl.BlockSpec((num_graphs, hidden), lambda: (0, 0)),
            pl.BlockSpec((num_graphs, 1), lambda: (0, 0)),
            pl.BlockSpec((hidden, 1), lambda: (0, 0)),
            pl.BlockSpec((1, 1), lambda: (0, 0)),
        ],
        out_specs=pl.BlockSpec((num_graphs, 1), lambda: (0, 0)),
        out_shape=jax.ShapeDtypeStruct((num_graphs, 1), f32),
    )

    xs = x[:, 0]
    p0 = jnp.zeros((n,), f32).at[dst].add(xs[src] * norm_e)[:, None]
    h = layer0(p0, x, selfw, W0, b0[None, :])

    for i in range(W_rest.shape[0]):
        agg = jnp.zeros((n, hidden), f32).at[dst].add(
            h[src] * norm_e[:, None])
        h = layer(agg, h, selfw, W_rest[i], b_rest[i][None, :])

    sums, cnt = pool(h, seg)
    out = head(sums, cnt, lin_W, lin_b[None, :])
    return out[:, 0]
